# baseline (speedup 1.0000x reference)
import jax
import jax.numpy as jnp
import numpy as np
from jax.experimental import pallas as pl
from jax.experimental.pallas import tpu as pltpu

_T, _H = 8, 8
_D = 1024
_N = 4096
_TS, _TN = 2048, 512

_PMAT = np.zeros((_T * _H, 128), np.float32)
for _t in range(_T):
    for _h in range(_H):
        _PMAT[_t * _H + _h, _t] = float(2 ** _h)
_PMAT_T8 = np.ascontiguousarray(_PMAT[:, :_T].T)


_C_ONES = np.int32(0x01010101)
_C_HIGH = np.int32(np.uint32(0x80808080).astype(np.int64) - (1 << 32))


def _pack4_cols(c):
    p0 = c[:, 0:1] | (c[:, 1:2] << 8) | (c[:, 2:3] << 16) | (c[:, 3:4] << 24)
    p1 = c[:, 4:5] | (c[:, 5:6] << 8) | (c[:, 6:7] << 16) | (c[:, 7:8] << 24)
    return p0, p1


def _body(x_ref, w_ref, b_ref, projT_ref, projM_ref, pmat_ref, pmatT8_ref,
          out_ref, hx_s, hw_s):
    i = pl.program_id(0)
    j = pl.program_id(1)

    @pl.when(j == 0)
    def _():
        dots = jnp.dot(x_ref[...], projT_ref[...],
                       preferred_element_type=jnp.float32)
        bits = (dots > 0).astype(jnp.float32)
        hx_s[...] = jnp.dot(bits, pmat_ref[...],
                            preferred_element_type=jnp.float32).astype(jnp.int32)

    @pl.when(i == 0)
    def _():
        dw = jax.lax.dot_general(projM_ref[...], w_ref[...],
                                 dimension_numbers=(((1,), (1,)), ((), ())),
                                 preferred_element_type=jnp.float32)
        bw = (dw > 0).astype(jnp.float32)
        hw_s[:, pl.ds(j * _TN, _TN)] = jnp.dot(
            pmatT8_ref[...], bw,
            preferred_element_type=jnp.float32).astype(jnp.int32)

    dense = jax.lax.dot_general(x_ref[...], w_ref[...],
                                dimension_numbers=(((1,), (1,)), ((), ())),
                                preferred_element_type=jnp.float32)
    cw = hw_s[:, pl.ds(j * _TN, _TN)]
    mask = hx_s[:, 0:1] == cw[0:1, :]
    for t in range(1, _T):
        mask = mask | (hx_s[:, t:t + 1] == cw[t:t + 1, :])
    out_ref[...] = jnp.where(mask, dense + b_ref[...], 0.0)


def kernel(x, W, b, proj):
    B, S, D = x.shape
    BS = B * S
    xf = x.reshape(BS, D)
    projM = proj.reshape(_T * _H, D)
    projT = projM.T
    b2 = b.reshape(1, _N)
    out = pl.pallas_call(
        _body,
        grid=(BS // _TS, _N // _TN),
        in_specs=[
            pl.BlockSpec((_TS, D), lambda i, j: (i, 0)),
            pl.BlockSpec((_TN, D), lambda i, j: (j, 0)),
            pl.BlockSpec((1, _TN), lambda i, j: (0, j)),
            pl.BlockSpec((D, _T * _H), lambda i, j: (0, 0)),
            pl.BlockSpec((_T * _H, D), lambda i, j: (0, 0)),
            pl.BlockSpec((_T * _H, 128), lambda i, j: (0, 0)),
            pl.BlockSpec((_T, _T * _H), lambda i, j: (0, 0)),
        ],
        out_specs=pl.BlockSpec((_TS, _TN), lambda i, j: (i, j)),
        out_shape=jax.ShapeDtypeStruct((BS, _N), jnp.float32),
        scratch_shapes=[
            pltpu.VMEM((_TS, 128), jnp.int32),
            pltpu.VMEM((_T, _N), jnp.int32),
        ],
    )(xf, W, b2, jnp.asarray(projT), projM, jnp.asarray(_PMAT),
      jnp.asarray(_PMAT_T8))
    return out.reshape(B, S, _N)
